# baseline (speedup 1.0000x reference)
import functools

import jax
import jax.numpy as jnp
from jax import lax
from jax.experimental import pallas as pl
from jax.experimental.pallas import tpu as pltpu
from jax.experimental.pallas import tpu_sc as plsc

_B, _T = 1024, 200
_D = 64
_N = _B * _T
_C = 128
_NCOLS = _D // 16

_info = plsc.get_sparse_core_info()
_NC, _NS = _info.num_cores, _info.num_subcores
_NW = _NC * _NS
_BPW = _N // _NW
_NCHUNK = _BPW // _C


def _pe_table():
    pos = jnp.arange(_T, dtype=jnp.float32)[:, None]
    div = jnp.power(10000.0, jnp.arange(0, _D, 2, dtype=jnp.float32) / _D)
    angle = pos / div
    pe = jnp.zeros((_T, _D), dtype=jnp.float32)
    pe = pe.at[:, 0::2].set(jnp.sin(angle))
    pe = pe.at[:, 1::2].set(jnp.cos(angle))
    return pe


def _body(table, idx, pe, out, idx_v, pe_v, rows_v, sem):
    cid = lax.axis_index("c")
    sid = lax.axis_index("s")
    wid = sid * _NC + cid

    pltpu.sync_copy(idx.at[wid], idx_v)
    pltpu.sync_copy(pe, pe_v)

    def chunk_body(c, carry):
        pltpu.async_copy(table.at[idx_v.at[c]], rows_v, sem).wait()

        def row_body(i, carry2):
            t = lax.rem(c * _C + i, _T)
            for col in range(_NCOLS):
                plsc.addupdate(
                    rows_v.at[i, pl.ds(col * 16, 16)],
                    pe_v[t, pl.ds(col * 16, 16)],
                )
            return carry2

        lax.fori_loop(0, _C, row_body, 0)
        pltpu.sync_copy(rows_v, out.at[pl.ds(wid * _BPW + c * _C, _C)])
        return carry

    lax.fori_loop(0, _NCHUNK, chunk_body, 0)


@functools.partial(jax.jit, static_argnames=())
def _run(x_flat2d, emb_weight, pe):
    mesh = plsc.VectorSubcoreMesh(core_axis_name="c", subcore_axis_name="s")
    k = pl.kernel(
        _body,
        mesh=mesh,
        out_type=jax.ShapeDtypeStruct((_N, _D), jnp.float32),
        scratch_types=[
            pltpu.VMEM((_NCHUNK, _C), jnp.int32),
            pltpu.VMEM((_T, _D), jnp.float32),
            pltpu.VMEM((_C, _D), jnp.float32),
            pltpu.SemaphoreType.DMA,
        ],
        compiler_params=pltpu.CompilerParams(use_tc_tiling_on_sc=False),
    )
    return k(emb_weight, x_flat2d, pe)


def kernel(x, emb_weight):
    x_flat2d = x.reshape(_NW, _NCHUNK, _C).astype(jnp.int32)
    out = _run(x_flat2d, emb_weight, _pe_table())
    return out.reshape(_B, _T, _D)


# speedup vs baseline: 1.0765x; 1.0765x over previous
import functools

import jax
import jax.numpy as jnp
from jax import lax
from jax.experimental import pallas as pl
from jax.experimental.pallas import tpu as pltpu
from jax.experimental.pallas import tpu_sc as plsc

_B, _T = 1024, 200
_D = 64
_N = _B * _T
_C = 128
_NCOLS = _D // 16

_info = plsc.get_sparse_core_info()
_NC, _NS = _info.num_cores, _info.num_subcores
_NW = _NC * _NS
_BPW = _N // _NW
_NCHUNK = _BPW // _C
_NBUF = 5
_NGRP = _NCHUNK // _NBUF


def _pe_table():
    pos = jnp.arange(_T, dtype=jnp.float32)[:, None]
    div = jnp.power(10000.0, jnp.arange(0, _D, 2, dtype=jnp.float32) / _D)
    angle = pos / div
    pe = jnp.zeros((_T, _D), dtype=jnp.float32)
    pe = pe.at[:, 0::2].set(jnp.sin(angle))
    pe = pe.at[:, 1::2].set(jnp.cos(angle))
    return pe


def _body(table, idx, pe, out, idx_v, pe_v, rows_v, gsem, osem):
    cid = lax.axis_index("c")
    sid = lax.axis_index("s")
    wid = sid * _NC + cid

    pltpu.sync_copy(idx.at[wid], idx_v)
    pltpu.sync_copy(pe, pe_v)

    def fire_group(g, ring):
        for b in range(_NBUF):
            pltpu.async_copy(
                table.at[idx_v.at[g * _NBUF + b]],
                rows_v.at[ring * _NBUF + b],
                gsem,
            )

    fire_group(0, 0)

    def group_body(g, carry):
        ring = lax.rem(g, 2)

        @pl.when(g >= 1)
        def _():
            for _b in range(_NBUF):
                pltpu.make_async_copy(
                    rows_v.at[0], out.at[pl.ds(0, _C)], osem
                ).wait()

        @pl.when(g <= _NGRP - 2)
        def _():
            fire_group(g + 1, lax.rem(g + 1, 2))

        for _b in range(_NBUF):
            pltpu.make_async_copy(
                table.at[idx_v.at[0]], rows_v.at[0], gsem
            ).wait()

        for b in range(_NBUF):
            buf = ring * _NBUF + b
            c = g * _NBUF + b

            def row_body(i, carry2):
                t = lax.rem(c * _C + i, _T)
                for col in range(_NCOLS):
                    plsc.addupdate(
                        rows_v.at[buf, i, pl.ds(col * 16, 16)],
                        pe_v[t, pl.ds(col * 16, 16)],
                    )
                return carry2

            lax.fori_loop(0, _C, row_body, 0)
            pltpu.async_copy(
                rows_v.at[buf],
                out.at[pl.ds(wid * _BPW + c * _C, _C)],
                osem,
            )
        return carry

    lax.fori_loop(0, _NGRP, group_body, 0)

    for _b in range(_NBUF):
        pltpu.make_async_copy(rows_v.at[0], out.at[pl.ds(0, _C)], osem).wait()


@functools.partial(jax.jit, static_argnames=())
def _run(x_flat2d, emb_weight, pe):
    mesh = plsc.VectorSubcoreMesh(core_axis_name="c", subcore_axis_name="s")
    k = pl.kernel(
        _body,
        mesh=mesh,
        out_type=jax.ShapeDtypeStruct((_N, _D), jnp.float32),
        scratch_types=[
            pltpu.VMEM((_NCHUNK, _C), jnp.int32),
            pltpu.VMEM((_T, _D), jnp.float32),
            pltpu.VMEM((2 * _NBUF, _C, _D), jnp.float32),
            pltpu.SemaphoreType.DMA,
            pltpu.SemaphoreType.DMA,
        ],
        compiler_params=pltpu.CompilerParams(use_tc_tiling_on_sc=False),
    )
    return k(emb_weight, x_flat2d, pe)


def kernel(x, emb_weight):
    x_flat2d = x.reshape(_NW, _NCHUNK, _C).astype(jnp.int32)
    out = _run(x_flat2d, emb_weight, _pe_table())
    return out.reshape(_B, _T, _D)
